# baseline (speedup 1.0000x reference)
import functools

import jax
import jax.numpy as jnp
from jax import lax
from jax.experimental import pallas as pl
from jax.experimental.pallas import tpu as pltpu
from jax.experimental.pallas import tpu_sc as plsc

_S = 64.0
_L_A = 10.0
_U_A = 110.0
_L_MARGIN = 0.45
_U_MARGIN = 0.8

_CC = 1408


def _margin_body(emb_ref, cos_ref, sin_ref, loss_ref):
    emb = emb_ref[...]
    xn = jnp.sqrt(jnp.sum(emb * emb, axis=1, keepdims=True))
    xn = jnp.clip(xn, _L_A, _U_A)
    ada = (_U_MARGIN - _L_MARGIN) / (_U_A - _L_A) * (xn - _L_A) + _L_MARGIN
    cos_ref[...] = jnp.cos(ada)
    sin_ref[...] = jnp.sin(ada)
    g = xn * (1.0 / (_U_A * _U_A)) + 1.0 / xn
    loss_ref[...] = jnp.sum(g).reshape(1, 1) / emb.shape[0]


def _nsqrt(x):
    i = lax.bitcast_convert_type(x, jnp.int32)
    y = lax.bitcast_convert_type(0x5F3759DF - (i >> 1), jnp.float32)
    for _ in range(3):
        y = y * (1.5 - 0.5 * x * y * y)
    return x * y


def _lane(vec, lane, fill):
    m = lax.iota(vec.dtype if vec.dtype == jnp.int32 else jnp.int32, 16) == lane
    return jnp.max(jnp.where(m, vec, fill))


def _sc_scale_patch(logits, labels, cos_m, sin_m, B, V):
    info = plsc.get_sparse_core_info()
    nw = info.num_cores * info.num_subcores
    ngrp = B // 8
    grp_pw = ngrp // nw
    rows_pw = grp_pw * 8
    nch = V // _CC
    if nch % 2:
        nch -= 1
    v_sc = nch * _CC
    mesh = plsc.VectorSubcoreMesh(core_axis_name="c", subcore_axis_name="s")

    @functools.partial(
        pl.kernel,
        out_type=jax.ShapeDtypeStruct((B, V), jnp.float32),
        mesh=mesh,
        scratch_types=[
            pltpu.VMEM((8, _CC), jnp.float32),
            pltpu.VMEM((8, _CC), jnp.float32),
            pltpu.VMEM((8, _CC), jnp.float32),
            pltpu.VMEM((8, _CC), jnp.float32),
            pltpu.VMEM((rows_pw,), jnp.int32),
            pltpu.VMEM((rows_pw,), jnp.float32),
            pltpu.VMEM((rows_pw,), jnp.float32),
            pltpu.SemaphoreType.DMA,
            pltpu.SemaphoreType.DMA,
            pltpu.SemaphoreType.DMA,
            pltpu.SemaphoreType.DMA,
        ],
    )
    def k(x_hbm, lab_hbm, cos_hbm, sin_hbm, o_hbm,
          in0, in1, ou0, ou1, lab_v, cos_v, sin_v, si0, si1, so0, so1):
        wid = lax.axis_index("s") * info.num_cores + lax.axis_index("c")
        base_row = wid * rows_pw
        ins = (in0, in1)
        outs = (ou0, ou1)
        sis = (si0, si1)
        sos = (so0, so1)

        pltpu.sync_copy(lab_hbm.at[pl.ds(base_row, rows_pw)], lab_v)
        pltpu.sync_copy(cos_hbm.at[pl.ds(base_row, rows_pw)], cos_v)
        pltpu.sync_copy(sin_hbm.at[pl.ds(base_row, rows_pw)], sin_v)

        for g_local in range(grp_pw):
            row0 = base_row + g_local * 8

            labs, coss, sins = [], [], []
            for rr in range(8):
                j = g_local * 8 + rr
                h = (j // 16) * 16
                lane = j % 16
                labs.append(lab_v[pl.ds(h, 16)][lane])
                coss.append(cos_v[pl.ds(h, 16)][lane])
                sins.append(sin_v[pl.ds(h, 16)][lane])

            def src(t):
                return x_hbm.at[pl.ds(row0, 8), pl.ds(t * _CC, _CC)]

            def dst(t):
                return o_hbm.at[pl.ds(row0, 8), pl.ds(t * _CC, _CC)]

            def chunk_work(t, b):
                pltpu.make_async_copy(src(t), ins[b], sis[b]).wait()

                @pl.when(t >= 2)
                def _():
                    pltpu.make_async_copy(outs[b], dst(t - 2), sos[b]).wait()

                def mul(kk, c, b=b):
                    for rr in range(8):
                        for u in range(2):
                            o = kk * 32 + u * 16
                            x16 = ins[b][rr, pl.ds(o, 16)]
                            outs[b][rr, pl.ds(o, 16)] = x16 * _S
                    return c

                lax.fori_loop(0, _CC // 32, mul, 0)

                c0 = t * _CC
                for rr in range(8):
                    lab = labs[rr]

                    @pl.when((lab >= c0) & (lab < c0 + _CC))
                    def _(b=b, rr=rr, lab=lab, cs=coss[rr], sn=sins[rr], c0=c0):
                        def psweep(kk, c):
                            o = pl.multiple_of(kk * 16, 16)
                            x16 = ins[b][rr, pl.ds(o, 16)]
                            m = lax.iota(jnp.int32, 16) + (c0 + o) == lab
                            sin_t = _nsqrt(jnp.maximum(1.0 - x16 * x16, 0.0))
                            nv16 = (x16 * cs - sin_t * sn) * _S
                            outs[b][rr, pl.ds(o, 16)] = jnp.where(
                                m, nv16, x16 * _S
                            )
                            return c

                        lax.fori_loop(0, _CC // 16, psweep, 0)

                pltpu.async_copy(outs[b], dst(t), sos[b])

                @pl.when(t + 2 < nch)
                def _():
                    pltpu.async_copy(src(t + 2), ins[b], sis[b])

            pltpu.async_copy(src(0), in0, si0)
            pltpu.async_copy(src(1), in1, si1)

            def step(i, carry):
                for b in range(2):
                    chunk_work(i * 2 + b, b)
                return carry

            lax.fori_loop(0, nch // 2, step, 0)
            pltpu.make_async_copy(ou0, dst(nch - 2), so0).wait()
            pltpu.make_async_copy(ou1, dst(nch - 1), so1).wait()

    return k(logits, labels, cos_m, sin_m)


def _stripe_body(v_sc, alias_ref, x_ref, lab_ref, cos_ref, sin_ref, o_ref):
    del alias_ref
    i = pl.program_id(0)
    j = pl.program_id(1)
    x = x_ref[...]
    o_ref[...] = x * _S
    c0 = v_sc + j * 128
    for rr in range(8):
        lab = lab_ref[i * 8 + rr]

        @pl.when((lab >= c0) & (lab < c0 + 128))
        def _(rr=rr, lab=lab):
            w = x_ref[pl.ds(rr, 1), :]
            m = lax.broadcasted_iota(jnp.int32, w.shape, 1) + c0 == lab
            sin_t = jnp.sqrt(jnp.maximum(1.0 - w * w, 0.0))
            nvw = (w * cos_ref[i * 8 + rr] - sin_t * sin_ref[i * 8 + rr]) * _S
            o_ref[pl.ds(rr, 1), :] = jnp.where(m, nvw, w * _S)


def kernel(logits, labels, embeddings):
    B, V = logits.shape
    labels = labels.astype(jnp.int32)

    cos_m, sin_m, loss = pl.pallas_call(
        _margin_body,
        out_shape=(
            jax.ShapeDtypeStruct((B, 1), jnp.float32),
            jax.ShapeDtypeStruct((B, 1), jnp.float32),
            jax.ShapeDtypeStruct((1, 1), jnp.float32),
        ),
        in_specs=[pl.BlockSpec(embeddings.shape, lambda: (0, 0))],
        out_specs=(
            pl.BlockSpec((B, 1), lambda: (0, 0)),
            pl.BlockSpec((B, 1), lambda: (0, 0)),
            pl.BlockSpec((1, 1), lambda: (0, 0)),
        ),
    )(embeddings)

    cos_f = cos_m.reshape(B)
    sin_f = sin_m.reshape(B)
    scaled = _sc_scale_patch(logits, labels, cos_f, sin_f, B, V)

    nch = V // _CC
    if nch % 2:
        nch -= 1
    v_sc = nch * _CC
    if v_sc < V:
        jtile = v_sc // 128
        ncb = pl.cdiv(V - v_sc, 128)
        scaled = pl.pallas_call(
            functools.partial(_stripe_body, v_sc),
            grid=(B // 8, ncb),
            in_specs=[
                pl.BlockSpec(memory_space=pl.ANY),
                pl.BlockSpec((8, 128), lambda i, j: (i, jtile + j)),
                pl.BlockSpec(memory_space=pltpu.SMEM),
                pl.BlockSpec(memory_space=pltpu.SMEM),
                pl.BlockSpec(memory_space=pltpu.SMEM),
            ],
            out_specs=pl.BlockSpec((8, 128), lambda i, j: (i, jtile + j)),
            out_shape=jax.ShapeDtypeStruct((B, V), jnp.float32),
            input_output_aliases={0: 0},
        )(scaled, logits, labels, cos_f, sin_f)

    return (scaled, loss.reshape(()))
